# baseline (speedup 1.0000x reference)
import jax
import jax.numpy as jnp
from jax.experimental import pallas as pl


_TM = 2048
_PACK = 8

_DN_T_RHS = (((1,), (1,)), ((), ()))
_DN_T_LHS = (((0,), (1,)), ((), ()))
_DN_COL = (((0,), (0,)), ((), ()))


def _proj_body(emb_ref, kp_ref, smt_ref, w_ref, b_ref, out_ref):
    emb_dim = emb_ref.shape[1]
    kp_dim = kp_ref.shape[1] // _PACK
    kp_lo = emb_dim + 5
    acc = jax.lax.dot_general(emb_ref[...], w_ref[:, :emb_dim], _DN_T_RHS,
                              preferred_element_type=jnp.float32)
    kp8 = kp_ref[...]
    kp_tile = jnp.stack(
        [kp8[:, j * kp_dim:(j + 1) * kp_dim] for j in range(_PACK)],
        axis=1).reshape(_TM, kp_dim)
    acc += jax.lax.dot_general(kp_tile, w_ref[:, kp_lo:], _DN_T_RHS,
                               preferred_element_type=jnp.float32)
    acc += jax.lax.dot_general(smt_ref[:5, :], w_ref[:, emb_dim:kp_lo],
                               _DN_T_LHS, preferred_element_type=jnp.float32)
    acc += b_ref[...]
    mcol = jax.lax.dot_general(smt_ref[5:6, :],
                               jnp.ones((1, 1), jnp.float32), _DN_COL,
                               preferred_element_type=jnp.float32)
    out_ref[...] = acc * mcol


def kernel(embeddings, visibility_scores, bbox_ltwh, keypoints_xyc, feats_masks, W, b):
    B, N = feats_masks.shape
    M = B * N
    emb_dim = embeddings.shape[-1]
    kp_dim = keypoints_xyc.shape[-2] * keypoints_xyc.shape[-1]
    token_dim = W.shape[0]

    emb = embeddings.reshape(M, emb_dim)
    kp8 = keypoints_xyc.reshape(M // _PACK, _PACK * kp_dim)
    smallT = jnp.concatenate(
        [visibility_scores.reshape(M, 1),
         bbox_ltwh.reshape(M, 4),
         feats_masks.reshape(M, 1).astype(jnp.float32)],
        axis=1).T
    b2 = b.reshape(1, token_dim)

    grid = (M // _TM,)
    out = pl.pallas_call(
        _proj_body,
        grid=grid,
        in_specs=[
            pl.BlockSpec((_TM, emb_dim), lambda i: (i, 0)),
            pl.BlockSpec((_TM // _PACK, _PACK * kp_dim), lambda i: (i, 0)),
            pl.BlockSpec((6, _TM), lambda i: (0, i)),
            pl.BlockSpec(W.shape, lambda i: (0, 0)),
            pl.BlockSpec(b2.shape, lambda i: (0, 0)),
        ],
        out_specs=pl.BlockSpec((_TM, token_dim), lambda i: (i, 0)),
        out_shape=jax.ShapeDtypeStruct((M, token_dim), jnp.float32),
    )(emb, kp8, smallT, W, b2)

    return out.reshape(B, N, token_dim)


# speedup vs baseline: 15.5342x; 15.5342x over previous
import jax
import jax.numpy as jnp
from jax.experimental import pallas as pl
from jax.experimental.pallas import tpu as pltpu


_TM = 2048

_DN_T_RHS = (((1,), (1,)), ((), ()))
_DN_T_LHS = (((0,), (1,)), ((), ()))
_DN_COL = (((0,), (0,)), ((), ()))


def _proj(emb_ref, smt_ref, w_ref, b_ref, out_ref):
    emb_dim = emb_ref.shape[1]
    n_small = smt_ref.shape[0] - 1
    acc = jax.lax.dot_general(emb_ref[...], w_ref[:, :emb_dim], _DN_T_RHS,
                              preferred_element_type=jnp.float32)
    acc += jax.lax.dot_general(smt_ref[:n_small, :], w_ref[:, emb_dim:],
                               _DN_T_LHS, preferred_element_type=jnp.float32)
    acc += b_ref[...]
    mcol = jax.lax.dot_general(smt_ref[n_small:, :],
                               jnp.ones((1, 1), jnp.float32), _DN_COL,
                               preferred_element_type=jnp.float32)
    out_ref[...] = acc * mcol


def _proj_first(emb_ref, smt_ref, w_ref, b_ref, out_ref):
    _proj(emb_ref, smt_ref, w_ref, b_ref, out_ref)


def _proj_second(emb_ref, smt_ref, w_ref, b_ref, prev_ref, out_ref):
    del prev_ref
    _proj(emb_ref, smt_ref, w_ref, b_ref, out_ref)


def _pack_small(vis, bbox, kp, msk):
    return jnp.concatenate(
        [vis, bbox, kp, msk.astype(jnp.float32)], axis=1).T


def kernel(embeddings, visibility_scores, bbox_ltwh, keypoints_xyc, feats_masks, W, b):
    B, N = feats_masks.shape
    M = B * N
    H = M // 2
    emb_dim = embeddings.shape[-1]
    kp_dim = keypoints_xyc.shape[-2] * keypoints_xyc.shape[-1]
    token_dim = W.shape[0]
    n_feat = kp_dim + 6

    emb = embeddings.reshape(M, emb_dim)
    vis = visibility_scores.reshape(M, 1)
    bbox = bbox_ltwh.reshape(M, 4)
    kp = keypoints_xyc.reshape(M, kp_dim)
    msk = feats_masks.reshape(M, 1)
    b2 = b.reshape(1, token_dim)

    smt_a = _pack_small(vis[:H], bbox[:H], kp[:H], msk[:H])
    smt_b = _pack_small(vis[H:], bbox[H:], kp[H:], msk[H:])

    n_half_blocks = H // _TM
    common_specs = [
        pl.BlockSpec((n_feat, _TM), lambda i: (0, i)),
        pl.BlockSpec(W.shape, lambda i: (0, 0)),
        pl.BlockSpec(b2.shape, lambda i: (0, 0)),
    ]
    out_shape = jax.ShapeDtypeStruct((M, token_dim), jnp.float32)

    out_a = pl.pallas_call(
        _proj_first,
        grid=(n_half_blocks,),
        in_specs=[pl.BlockSpec((_TM, emb_dim), lambda i: (i, 0))] + common_specs,
        out_specs=pl.BlockSpec((_TM, token_dim), lambda i: (i, 0)),
        out_shape=out_shape,
    )(emb, smt_a, W, b2)

    off = n_half_blocks
    out = pl.pallas_call(
        _proj_second,
        grid=(n_half_blocks,),
        in_specs=[pl.BlockSpec((_TM, emb_dim), lambda i: (i + off, 0))]
        + common_specs
        + [pl.BlockSpec(memory_space=pltpu.MemorySpace.HBM)],
        out_specs=pl.BlockSpec((_TM, token_dim), lambda i: (i + off, 0)),
        out_shape=out_shape,
        input_output_aliases={4: 0},
    )(emb, smt_b, W, b2, out_a)

    return out.reshape(B, N, token_dim)


# speedup vs baseline: 20.4639x; 1.3173x over previous
import jax
import jax.numpy as jnp
from jax.experimental import pallas as pl


_TM = 2048

_DN_T_RHS = (((1,), (1,)), ((), ()))
_DN_T_LHS = (((0,), (1,)), ((), ()))
_DN_COL = (((0,), (0,)), ((), ()))


def _proj_body(emb_ref, smt_ref, w_ref, b_ref, out_ref):
    emb_dim = emb_ref.shape[1]
    n_small = smt_ref.shape[0] - 1
    acc = jax.lax.dot_general(emb_ref[...], w_ref[:, :emb_dim], _DN_T_RHS,
                              preferred_element_type=jnp.float32)
    w_small = w_ref[:, emb_dim:].astype(jnp.bfloat16)
    acc += jax.lax.dot_general(smt_ref[:n_small, :], w_small, _DN_T_LHS,
                               preferred_element_type=jnp.float32)
    acc += b_ref[...]
    mcol = jax.lax.dot_general(smt_ref[n_small:, :],
                               jnp.ones((1, 1), jnp.bfloat16), _DN_COL,
                               preferred_element_type=jnp.float32)
    out_ref[...] = acc * mcol


def kernel(embeddings, visibility_scores, bbox_ltwh, keypoints_xyc, feats_masks, W, b):
    B, N = feats_masks.shape
    M = B * N
    emb_dim = embeddings.shape[-1]
    kp_dim = keypoints_xyc.shape[-2] * keypoints_xyc.shape[-1]
    token_dim = W.shape[0]

    emb = embeddings.reshape(M, emb_dim)
    smallT = jnp.concatenate(
        [visibility_scores.reshape(M, 1),
         bbox_ltwh.reshape(M, 4),
         keypoints_xyc.reshape(M, kp_dim),
         feats_masks.reshape(M, 1).astype(jnp.float32)],
        axis=1).astype(jnp.bfloat16).T
    b2 = b.reshape(1, token_dim)

    grid = (M // _TM,)
    out = pl.pallas_call(
        _proj_body,
        grid=grid,
        in_specs=[
            pl.BlockSpec((_TM, emb_dim), lambda i: (i, 0)),
            pl.BlockSpec((kp_dim + 6, _TM), lambda i: (0, i)),
            pl.BlockSpec(W.shape, lambda i: (0, 0)),
            pl.BlockSpec(b2.shape, lambda i: (0, 0)),
        ],
        out_specs=pl.BlockSpec((_TM, token_dim), lambda i: (i, 0)),
        out_shape=jax.ShapeDtypeStruct((M, token_dim), jnp.float32),
    )(emb, smallT, W, b2)

    return out.reshape(B, N, token_dim)
